# baseline (speedup 1.0000x reference)
import functools

import jax
import jax.numpy as jnp
from jax import lax
from jax.experimental import pallas as pl
from jax.experimental.pallas import tpu as pltpu
from jax.experimental.pallas import tpu_sc as plsc

N = 10000
E = 320000
D_IN = 128
HID = 64
P = 50000

NR = 10240
NC, NS = 2, 16
TILES = NC * NS
CH = 128
ROWS = 80
EPAD = TILES * ROWS * CH
SLICE = NR // NS
PPT = 1600
PPAD = TILES * PPT
DW = 8


@functools.cache
def _sc_kernels():
    mesh = plsc.VectorSubcoreMesh(core_axis_name="c", subcore_axis_name="s")
    sc_params = pltpu.CompilerParams(use_tc_tiling_on_sc=False,
                                     needs_layout_passes=False)
    sc_params_tc_tiled = pltpu.CompilerParams(needs_layout_passes=False)

    @functools.partial(
        pl.kernel,
        mesh=mesh,
        out_type=[jax.ShapeDtypeStruct((NR,), jnp.float32),
                  jax.ShapeDtypeStruct((NR,), jnp.float32)],
        compiler_params=sc_params,
        scratch_types=[
            pltpu.VMEM((ROWS, CH), jnp.int32),
            pltpu.VMEM((CH,), jnp.float32),
            pltpu.VMEM_SHARED((NR,), jnp.float32),
            pltpu.SemaphoreType.DMA,
        ],
    )
    def deg_kernel(edges_hbm, ones_hbm, zeros_hbm, out0_hbm, out1_hbm,
                   idx_v, ones_v, acc, sem):
        cid = lax.axis_index("c")
        sid = lax.axis_index("s")
        g = cid * NS + sid
        pltpu.sync_copy(zeros_hbm.at[pl.ds(sid * SLICE, SLICE)],
                        acc.at[pl.ds(sid * SLICE, SLICE)])
        pltpu.sync_copy(ones_hbm, ones_v)
        pltpu.sync_copy(edges_hbm.at[1, pl.ds(g * ROWS, ROWS)], idx_v)
        plsc.subcore_barrier()

        @pl.loop(0, ROWS)
        def _(j):
            pltpu.async_copy(ones_v, acc.at[idx_v.at[j]], sem, add=True)

        @pl.loop(0, ROWS)
        def _(j):
            pltpu.make_async_copy(ones_v, acc.at[idx_v.at[j]], sem).wait()

        plsc.subcore_barrier()

        @pl.when(cid == 0)
        def _():
            pltpu.sync_copy(acc.at[pl.ds(sid * SLICE, SLICE)],
                            out0_hbm.at[pl.ds(sid * SLICE, SLICE)])

        @pl.when(cid == 1)
        def _():
            pltpu.sync_copy(acc.at[pl.ds(sid * SLICE, SLICE)],
                            out1_hbm.at[pl.ds(sid * SLICE, SLICE)])

    @functools.partial(
        pl.kernel,
        mesh=mesh,
        out_type=[jax.ShapeDtypeStruct((NR, HID), jnp.float32),
                  jax.ShapeDtypeStruct((NR, HID), jnp.float32)],
        compiler_params=sc_params,
        scratch_types=[
            pltpu.VMEM((ROWS, CH), jnp.int32),
            pltpu.VMEM((ROWS, CH), jnp.int32),
            [pltpu.VMEM((CH, HID), jnp.float32)] * 8,
            [pltpu.SemaphoreType.DMA] * 8,
            [pltpu.SemaphoreType.DMA] * 8,
            pltpu.VMEM_SHARED((NR, HID), jnp.float32),
        ],
    )
    def scatter_kernel(edges_hbm, m_hbm, zeros_hbm, out0_hbm, out1_hbm,
                       isrc, idst, bufs, gsems, ssems, acc):
        cid = lax.axis_index("c")
        sid = lax.axis_index("s")
        g = cid * NS + sid
        pltpu.sync_copy(zeros_hbm.at[pl.ds(sid * SLICE, SLICE)],
                        acc.at[pl.ds(sid * SLICE, SLICE)])
        pltpu.sync_copy(edges_hbm.at[0, pl.ds(g * ROWS, ROWS)], isrc)
        pltpu.sync_copy(edges_hbm.at[1, pl.ds(g * ROWS, ROWS)], idst)
        plsc.subcore_barrier()

        def gather(j, k):
            pltpu.async_copy(m_hbm.at[isrc.at[j]], bufs[k], gsems[k])

        def gather_wait(j, k):
            pltpu.make_async_copy(m_hbm.at[isrc.at[j]], bufs[k],
                                  gsems[k]).wait()

        def scat(j, k):
            pltpu.async_copy(bufs[k], acc.at[idst.at[j]], ssems[k], add=True)

        def scat_wait(j, k):
            pltpu.make_async_copy(bufs[k], acc.at[idst.at[j]],
                                  ssems[k]).wait()

        nb = 8
        for k in range(nb):
            gather(k, k)

        @pl.loop(0, ROWS // nb)
        def _(jj):
            j = jj * nb
            for k in range(nb):
                gather_wait(j + k, k)
                scat(j + k, k)
                kp = (k + nb // 2) % nb
                if k < nb // 2:
                    @pl.when(jj > 0)
                    def _():
                        scat_wait(j + k - nb // 2, kp)
                        gather(j + k + nb // 2, kp)
                else:
                    scat_wait(j + k - nb // 2, kp)

                    @pl.when(jj < ROWS // nb - 1)
                    def _():
                        gather(j + k + nb // 2, kp)

        for k in range(nb // 2, nb):
            scat_wait(ROWS - nb + k, k)
        plsc.subcore_barrier()

        @pl.when(cid == 0)
        def _():
            pltpu.sync_copy(acc.at[pl.ds(sid * SLICE, SLICE)],
                            out0_hbm.at[pl.ds(sid * SLICE, SLICE)])

        @pl.when(cid == 1)
        def _():
            pltpu.sync_copy(acc.at[pl.ds(sid * SLICE, SLICE)],
                            out1_hbm.at[pl.ds(sid * SLICE, SLICE)])

    @functools.partial(
        pl.kernel,
        mesh=mesh,
        out_type=jax.ShapeDtypeStruct((P,), jnp.float32),
        compiler_params=sc_params_tc_tiled,
        scratch_types=[
            pltpu.VMEM((2 * PPT,), jnp.int32),
            pltpu.VMEM((2 * NR,), jnp.float32),
            pltpu.VMEM((PPT,), jnp.float32),
        ],
    )
    def pairs_kernel(pv_hbm, w_hbm, out_hbm, pv_v, w_v, out_v):
        cid = lax.axis_index("c")
        sid = lax.axis_index("s")
        g = cid * NS + sid
        pltpu.sync_copy(w_hbm, w_v)
        pltpu.sync_copy(pv_hbm.at[pl.ds(g * 2 * PPT, 2 * PPT)], pv_v)
        iota2 = 2 * lax.iota(jnp.int32, 16)

        @pl.loop(0, PPT // 16)
        def _(i):
            base = 32 * i + iota2
            ia = plsc.load_gather(pv_v, [base])
            ib = plsc.load_gather(pv_v, [base + 1]) + NR
            va = plsc.load_gather(w_v, [ia])
            vb = plsc.load_gather(w_v, [ib])
            out_v[pl.ds(i * 16, 16)] = va + vb

        @pl.when(g < TILES - 1)
        def _():
            pltpu.sync_copy(out_v, out_hbm.at[pl.ds(g * PPT, PPT)])

        @pl.when(g == TILES - 1)
        def _():
            pltpu.sync_copy(out_v.at[pl.ds(0, P - (TILES - 1) * PPT)],
                            out_hbm.at[pl.ds(g * PPT, P - (TILES - 1) * PPT)])

    return deg_kernel, scatter_kernel, pairs_kernel


BLK = 2048


def _dinv(d0_ref, d1_ref):
    return lax.rsqrt(d0_ref[...] + d1_ref[...] + 1.0)[:, None]


def _k1_body(d0_ref, d1_ref, x_ref, w_ref, m_ref):
    h = jnp.dot(x_ref[...], w_ref[...], preferred_element_type=jnp.float32)
    m_ref[...] = _dinv(d0_ref, d1_ref) * h


def _k2_body(d0_ref, d1_ref, p0_ref, p1_ref, m_ref, b_ref, w_ref, out_ref):
    dinv = _dinv(d0_ref, d1_ref)
    s = p0_ref[...] + p1_ref[...] + m_ref[...]
    h = jnp.maximum(dinv * s + b_ref[...], 0.0)
    out_ref[...] = dinv * jnp.dot(h, w_ref[...],
                                  preferred_element_type=jnp.float32)


def _k3_body(d0_ref, d1_ref, p0_ref, p1_ref, m_ref, b_ref, w_ref, bias_ref,
             z_ref):
    dinv = _dinv(d0_ref, d1_ref)
    s = p0_ref[...] + p1_ref[...] + m_ref[...]
    h = jnp.maximum(dinv * s + b_ref[...], 0.0)
    z = jnp.dot(h, w_ref[...], preferred_element_type=jnp.float32)
    z_ref[...] = z.T + bias_ref[...]


def _row_spec(w):
    return pl.BlockSpec((BLK, w), lambda i: (i, 0))


def _vec_spec():
    return pl.BlockSpec((BLK,), lambda i: (i,))


def _full_spec(r, c):
    return pl.BlockSpec((r, c), lambda i: (0, 0))


_GRID = NR // BLK

_k1 = pl.pallas_call(
    _k1_body,
    grid=(_GRID,),
    in_specs=[_vec_spec(), _vec_spec(), _row_spec(D_IN),
              _full_spec(D_IN, HID)],
    out_specs=_row_spec(HID),
    out_shape=jax.ShapeDtypeStruct((NR, HID), jnp.float32),
)

_k2 = pl.pallas_call(
    _k2_body,
    grid=(_GRID,),
    in_specs=[_vec_spec(), _vec_spec(), _row_spec(HID), _row_spec(HID),
              _row_spec(HID), _full_spec(1, HID), _full_spec(HID, HID)],
    out_specs=_row_spec(HID),
    out_shape=jax.ShapeDtypeStruct((NR, HID), jnp.float32),
)

_k3 = pl.pallas_call(
    _k3_body,
    grid=(_GRID,),
    in_specs=[_vec_spec(), _vec_spec(), _row_spec(HID), _row_spec(HID),
              _row_spec(HID), _full_spec(1, HID), _full_spec(HID, 2),
              _full_spec(2, 1)],
    out_specs=pl.BlockSpec((2, BLK), lambda i: (0, i)),
    out_shape=jax.ShapeDtypeStruct((2, NR), jnp.float32),
)


def kernel(x, edge_index, batch, pairs, W1, b1, W2, b2, Wh, bh):
    del batch
    i32 = jnp.int32
    f32 = jnp.float32
    deg_kernel, scatter_kernel, pairs_kernel = _sc_kernels()

    pad_idx = (jnp.arange(EPAD - E, dtype=i32) % (NR - N)) + N
    pad2 = jnp.broadcast_to(pad_idx, (2, EPAD - E))
    edges = jnp.concatenate([edge_index.astype(i32), pad2],
                            axis=1).reshape(2, TILES * ROWS, CH)
    edges = lax.optimization_barrier(edges)

    xp = jnp.pad(x.astype(f32), ((0, NR - N), (0, 0)))
    ones_deg = jnp.ones((CH,), f32)
    zeros_deg = jnp.zeros((NR,), f32)
    zeros_h = jnp.zeros((NR, HID), f32)

    pvt = jnp.pad(pairs.astype(i32), ((0, PPAD - P), (0, 0))).reshape(2 * PPAD)

    wh2 = jnp.stack([Wh[:HID, 0], Wh[HID:, 0]], axis=1).astype(f32)
    bias2 = jnp.concatenate([jnp.zeros((1,), f32), bh.astype(f32)])[:, None]

    d0, d1 = deg_kernel(edges, ones_deg, zeros_deg)
    m1 = _k1(d0, d1, xp, W1.astype(f32))
    p0, p1 = scatter_kernel(edges, m1, zeros_h)
    m2 = _k2(d0, d1, p0, p1, m1, b1.astype(f32)[None, :], W2.astype(f32))
    q0, q1 = scatter_kernel(edges, m2, zeros_h)
    z = _k3(d0, d1, q0, q1, m2, b2.astype(f32)[None, :], wh2, bias2)
    return pairs_kernel(pvt, z.reshape(2 * NR))


# speedup vs baseline: 1.1021x; 1.1021x over previous
import functools

import jax
import jax.numpy as jnp
from jax import lax
from jax.experimental import pallas as pl
from jax.experimental.pallas import tpu as pltpu
from jax.experimental.pallas import tpu_sc as plsc

N = 10000
E = 320000
D_IN = 128
HID = 64
P = 50000

NR = 10240
NC, NS = 2, 16
TILES = NC * NS
CH = 128
ROWS = 80
EPAD = TILES * ROWS * CH
SLICE = NR // NS
PPT = 1600
PPAD = TILES * PPT
DW = 8


@functools.cache
def _sc_kernels():
    mesh = plsc.VectorSubcoreMesh(core_axis_name="c", subcore_axis_name="s")
    sc_params = pltpu.CompilerParams(use_tc_tiling_on_sc=False,
                                     needs_layout_passes=False)
    sc_params_tc_tiled = pltpu.CompilerParams(needs_layout_passes=False)

    @functools.partial(
        pl.kernel,
        mesh=mesh,
        out_type=[jax.ShapeDtypeStruct((NR,), jnp.float32),
                  jax.ShapeDtypeStruct((NR,), jnp.float32)],
        compiler_params=sc_params,
        scratch_types=[
            pltpu.VMEM((ROWS, CH), jnp.int32),
            pltpu.VMEM((CH,), jnp.float32),
            pltpu.VMEM_SHARED((NR,), jnp.float32),
            pltpu.SemaphoreType.DMA,
        ],
    )
    def deg_kernel(edges_hbm, ones_hbm, zeros_hbm, out0_hbm, out1_hbm,
                   idx_v, ones_v, acc, sem):
        cid = lax.axis_index("c")
        sid = lax.axis_index("s")
        g = cid * NS + sid
        pltpu.sync_copy(zeros_hbm.at[pl.ds(sid * SLICE, SLICE)],
                        acc.at[pl.ds(sid * SLICE, SLICE)])
        pltpu.sync_copy(ones_hbm, ones_v)
        pltpu.sync_copy(edges_hbm.at[1, pl.ds(g * ROWS, ROWS)], idx_v)
        plsc.subcore_barrier()

        @pl.loop(0, ROWS)
        def _(j):
            pltpu.async_copy(ones_v, acc.at[idx_v.at[j]], sem, add=True)

        @pl.loop(0, ROWS)
        def _(j):
            pltpu.make_async_copy(ones_v, acc.at[idx_v.at[j]], sem).wait()

        plsc.subcore_barrier()

        @pl.when(cid == 0)
        def _():
            pltpu.sync_copy(acc.at[pl.ds(sid * SLICE, SLICE)],
                            out0_hbm.at[pl.ds(sid * SLICE, SLICE)])

        @pl.when(cid == 1)
        def _():
            pltpu.sync_copy(acc.at[pl.ds(sid * SLICE, SLICE)],
                            out1_hbm.at[pl.ds(sid * SLICE, SLICE)])

    @functools.partial(
        pl.kernel,
        mesh=mesh,
        out_type=[jax.ShapeDtypeStruct((NR, HID), jnp.float32),
                  jax.ShapeDtypeStruct((NR, HID), jnp.float32)],
        compiler_params=sc_params,
        scratch_types=[
            pltpu.VMEM((ROWS, CH), jnp.int32),
            pltpu.VMEM((ROWS, CH), jnp.int32),
            [pltpu.VMEM((CH, HID), jnp.float32)] * 8,
            [pltpu.SemaphoreType.DMA] * 8,
            [pltpu.SemaphoreType.DMA] * 8,
            pltpu.VMEM_SHARED((NR, HID), jnp.float32),
        ],
    )
    def scatter_kernel(edges_hbm, m_hbm, zeros_hbm, out0_hbm, out1_hbm,
                       isrc, idst, bufs, gsems, ssems, acc):
        cid = lax.axis_index("c")
        sid = lax.axis_index("s")
        g = cid * NS + sid
        pltpu.sync_copy(zeros_hbm.at[pl.ds(sid * SLICE, SLICE)],
                        acc.at[pl.ds(sid * SLICE, SLICE)])
        pltpu.sync_copy(edges_hbm.at[0, pl.ds(g * ROWS, ROWS)], isrc)
        pltpu.sync_copy(edges_hbm.at[1, pl.ds(g * ROWS, ROWS)], idst)
        plsc.subcore_barrier()

        def gather(j, k):
            pltpu.async_copy(m_hbm.at[isrc.at[j]], bufs[k], gsems[k])

        def gather_wait(j, k):
            pltpu.make_async_copy(m_hbm.at[isrc.at[j]], bufs[k],
                                  gsems[k]).wait()

        def scat(j, k):
            pltpu.async_copy(bufs[k], acc.at[idst.at[j]], ssems[k], add=True)

        def scat_wait(j, k):
            pltpu.make_async_copy(bufs[k], acc.at[idst.at[j]],
                                  ssems[k]).wait()

        nb = 8
        for k in range(nb):
            gather(k, k)

        @pl.loop(0, ROWS // nb)
        def _(jj):
            j = jj * nb
            for k in range(nb):
                gather_wait(j + k, k)
                scat(j + k, k)
                kp = (k + nb // 2) % nb
                if k < nb // 2:
                    @pl.when(jj > 0)
                    def _():
                        scat_wait(j + k - nb // 2, kp)
                        gather(j + k + nb // 2, kp)
                else:
                    scat_wait(j + k - nb // 2, kp)

                    @pl.when(jj < ROWS // nb - 1)
                    def _():
                        gather(j + k + nb // 2, kp)

        for k in range(nb // 2, nb):
            scat_wait(ROWS - nb + k, k)
        plsc.subcore_barrier()

        @pl.when(cid == 0)
        def _():
            pltpu.sync_copy(acc.at[pl.ds(sid * SLICE, SLICE)],
                            out0_hbm.at[pl.ds(sid * SLICE, SLICE)])

        @pl.when(cid == 1)
        def _():
            pltpu.sync_copy(acc.at[pl.ds(sid * SLICE, SLICE)],
                            out1_hbm.at[pl.ds(sid * SLICE, SLICE)])

    @functools.partial(
        pl.kernel,
        mesh=mesh,
        out_type=jax.ShapeDtypeStruct((P,), jnp.float32),
        compiler_params=sc_params_tc_tiled,
        scratch_types=[
            pltpu.VMEM((PPT,), jnp.int32),
            pltpu.VMEM((PPT,), jnp.int32),
            pltpu.VMEM((2 * NR,), jnp.float32),
            pltpu.VMEM((PPT,), jnp.float32),
        ],
    )
    def pairs_kernel(pv_hbm, w_hbm, out_hbm, pa_v, pb_v, w_v, out_v):
        cid = lax.axis_index("c")
        sid = lax.axis_index("s")
        g = cid * NS + sid
        pltpu.sync_copy(w_hbm, w_v)
        pltpu.sync_copy(pv_hbm.at[pl.ds(g * PPT, PPT)], pa_v)
        pltpu.sync_copy(pv_hbm.at[pl.ds(PPAD + g * PPT, PPT)], pb_v)

        @pl.loop(0, PPT // 16)
        def _(i):
            ia = pa_v[pl.ds(i * 16, 16)]
            ib = pb_v[pl.ds(i * 16, 16)] + NR
            va = plsc.load_gather(w_v, [ia])
            vb = plsc.load_gather(w_v, [ib])
            out_v[pl.ds(i * 16, 16)] = va + vb

        @pl.when(g < TILES - 1)
        def _():
            pltpu.sync_copy(out_v, out_hbm.at[pl.ds(g * PPT, PPT)])

        @pl.when(g == TILES - 1)
        def _():
            pltpu.sync_copy(out_v.at[pl.ds(0, P - (TILES - 1) * PPT)],
                            out_hbm.at[pl.ds(g * PPT, P - (TILES - 1) * PPT)])

    return deg_kernel, scatter_kernel, pairs_kernel


BLK = 2048


def _dinv(d0_ref, d1_ref):
    return lax.rsqrt(d0_ref[...] + d1_ref[...] + 1.0)[:, None]


def _k1_body(d0_ref, d1_ref, x_ref, w_ref, m_ref):
    h = jnp.dot(x_ref[...], w_ref[...], preferred_element_type=jnp.float32)
    m_ref[...] = _dinv(d0_ref, d1_ref) * h


def _k2_body(d0_ref, d1_ref, p0_ref, p1_ref, m_ref, b_ref, w_ref, out_ref):
    dinv = _dinv(d0_ref, d1_ref)
    s = p0_ref[...] + p1_ref[...] + m_ref[...]
    h = jnp.maximum(dinv * s + b_ref[...], 0.0)
    out_ref[...] = dinv * jnp.dot(h, w_ref[...],
                                  preferred_element_type=jnp.float32)


def _k3_body(d0_ref, d1_ref, p0_ref, p1_ref, m_ref, b_ref, w_ref, bias_ref,
             z_ref):
    dinv = _dinv(d0_ref, d1_ref)
    s = p0_ref[...] + p1_ref[...] + m_ref[...]
    h = jnp.maximum(dinv * s + b_ref[...], 0.0)
    z = jnp.dot(h, w_ref[...], preferred_element_type=jnp.float32)
    z_ref[...] = z.T + bias_ref[...]


def _row_spec(w):
    return pl.BlockSpec((BLK, w), lambda i: (i, 0))


def _vec_spec():
    return pl.BlockSpec((BLK,), lambda i: (i,))


def _full_spec(r, c):
    return pl.BlockSpec((r, c), lambda i: (0, 0))


_GRID = NR // BLK

_k1 = pl.pallas_call(
    _k1_body,
    grid=(_GRID,),
    in_specs=[_vec_spec(), _vec_spec(), _row_spec(D_IN),
              _full_spec(D_IN, HID)],
    out_specs=_row_spec(HID),
    out_shape=jax.ShapeDtypeStruct((NR, HID), jnp.float32),
)

_k2 = pl.pallas_call(
    _k2_body,
    grid=(_GRID,),
    in_specs=[_vec_spec(), _vec_spec(), _row_spec(HID), _row_spec(HID),
              _row_spec(HID), _full_spec(1, HID), _full_spec(HID, HID)],
    out_specs=_row_spec(HID),
    out_shape=jax.ShapeDtypeStruct((NR, HID), jnp.float32),
)

_k3 = pl.pallas_call(
    _k3_body,
    grid=(_GRID,),
    in_specs=[_vec_spec(), _vec_spec(), _row_spec(HID), _row_spec(HID),
              _row_spec(HID), _full_spec(1, HID), _full_spec(HID, 2),
              _full_spec(2, 1)],
    out_specs=pl.BlockSpec((2, BLK), lambda i: (0, i)),
    out_shape=jax.ShapeDtypeStruct((2, NR), jnp.float32),
)


def kernel(x, edge_index, batch, pairs, W1, b1, W2, b2, Wh, bh):
    del batch
    i32 = jnp.int32
    f32 = jnp.float32
    deg_kernel, scatter_kernel, pairs_kernel = _sc_kernels()

    pad_idx = (jnp.arange(EPAD - E, dtype=i32) % (NR - N)) + N
    pad2 = jnp.broadcast_to(pad_idx, (2, EPAD - E))
    edges = jnp.concatenate([edge_index.astype(i32), pad2],
                            axis=1).reshape(2, TILES * ROWS, CH)
    edges = lax.optimization_barrier(edges)

    xp = jnp.pad(x.astype(f32), ((0, NR - N), (0, 0)))
    ones_deg = jnp.ones((CH,), f32)
    zeros_deg = jnp.zeros((NR,), f32)
    zeros_h = jnp.zeros((NR, HID), f32)

    pvt = jnp.concatenate([
        jnp.pad(pairs[:, 0].astype(i32), (0, PPAD - P)),
        jnp.pad(pairs[:, 1].astype(i32), (0, PPAD - P)),
    ])

    wh2 = jnp.stack([Wh[:HID, 0], Wh[HID:, 0]], axis=1).astype(f32)
    bias2 = jnp.concatenate([jnp.zeros((1,), f32), bh.astype(f32)])[:, None]

    d0, d1 = deg_kernel(edges, ones_deg, zeros_deg)
    m1 = _k1(d0, d1, xp, W1.astype(f32))
    p0, p1 = scatter_kernel(edges, m1, zeros_h)
    m2 = _k2(d0, d1, p0, p1, m1, b1.astype(f32)[None, :], W2.astype(f32))
    q0, q1 = scatter_kernel(edges, m2, zeros_h)
    z = _k3(d0, d1, q0, q1, m2, b2.astype(f32)[None, :], wh2, bias2)
    return pairs_kernel(pvt, z.reshape(2 * NR))


# speedup vs baseline: 1.1339x; 1.0288x over previous
import functools

import jax
import jax.numpy as jnp
from jax import lax
from jax.experimental import pallas as pl
from jax.experimental.pallas import tpu as pltpu
from jax.experimental.pallas import tpu_sc as plsc

N = 10000
E = 320000
D_IN = 128
HID = 64
P = 50000

NR = 10240
NC, NS = 2, 16
TILES = NC * NS
CH = 128
ROWS = 80
EPAD = TILES * ROWS * CH
SLICE = NR // NS
PPT = 1600
PPAD = TILES * PPT
DW = 8


@functools.cache
def _sc_kernels():
    mesh = plsc.VectorSubcoreMesh(core_axis_name="c", subcore_axis_name="s")
    sc_params = pltpu.CompilerParams(use_tc_tiling_on_sc=False,
                                     needs_layout_passes=False)
    sc_params_tc_tiled = pltpu.CompilerParams(needs_layout_passes=False)

    @functools.partial(
        pl.kernel,
        mesh=mesh,
        out_type=[jax.ShapeDtypeStruct((NR,), jnp.float32),
                  jax.ShapeDtypeStruct((NR,), jnp.float32)],
        compiler_params=sc_params,
        scratch_types=[
            pltpu.VMEM((ROWS, CH), jnp.int32),
            pltpu.VMEM((CH,), jnp.float32),
            pltpu.VMEM_SHARED((NR,), jnp.float32),
            pltpu.SemaphoreType.DMA,
        ],
    )
    def deg_kernel(edges_hbm, ones_hbm, zeros_hbm, out0_hbm, out1_hbm,
                   idx_v, ones_v, acc, sem):
        cid = lax.axis_index("c")
        sid = lax.axis_index("s")
        g = cid * NS + sid
        pltpu.sync_copy(zeros_hbm.at[pl.ds(sid * SLICE, SLICE)],
                        acc.at[pl.ds(sid * SLICE, SLICE)])
        pltpu.sync_copy(ones_hbm, ones_v)
        pltpu.sync_copy(edges_hbm.at[1, pl.ds(g * ROWS, ROWS)], idx_v)
        plsc.subcore_barrier()

        @pl.loop(0, ROWS)
        def _(j):
            pltpu.async_copy(ones_v, acc.at[idx_v.at[j]], sem, add=True)

        @pl.loop(0, ROWS)
        def _(j):
            pltpu.make_async_copy(ones_v, acc.at[idx_v.at[j]], sem).wait()

        plsc.subcore_barrier()

        @pl.when(cid == 0)
        def _():
            pltpu.sync_copy(acc.at[pl.ds(sid * SLICE, SLICE)],
                            out0_hbm.at[pl.ds(sid * SLICE, SLICE)])

        @pl.when(cid == 1)
        def _():
            pltpu.sync_copy(acc.at[pl.ds(sid * SLICE, SLICE)],
                            out1_hbm.at[pl.ds(sid * SLICE, SLICE)])

    @functools.partial(
        pl.kernel,
        mesh=mesh,
        out_type=[jax.ShapeDtypeStruct((NR, HID), jnp.float32),
                  jax.ShapeDtypeStruct((NR, HID), jnp.float32)],
        compiler_params=sc_params,
        scratch_types=[
            pltpu.VMEM((ROWS, CH), jnp.int32),
            pltpu.VMEM((ROWS, CH), jnp.int32),
            [pltpu.VMEM((CH, HID), jnp.float32)] * 8,
            [pltpu.SemaphoreType.DMA] * 8,
            [pltpu.SemaphoreType.DMA] * 8,
            pltpu.VMEM_SHARED((NR, HID), jnp.float32),
        ],
    )
    def scatter_kernel(edges_hbm, m_hbm, zeros_hbm, out0_hbm, out1_hbm,
                       isrc, idst, bufs, gsems, ssems, acc):
        cid = lax.axis_index("c")
        sid = lax.axis_index("s")
        g = cid * NS + sid
        pltpu.sync_copy(zeros_hbm.at[pl.ds(sid * SLICE, SLICE)],
                        acc.at[pl.ds(sid * SLICE, SLICE)])
        pltpu.sync_copy(edges_hbm.at[0, pl.ds(g * ROWS, ROWS)], isrc)
        pltpu.sync_copy(edges_hbm.at[1, pl.ds(g * ROWS, ROWS)], idst)
        plsc.subcore_barrier()

        def gather(j, k):
            pltpu.async_copy(m_hbm.at[isrc.at[j]], bufs[k], gsems[k])

        def gather_wait(j, k):
            pltpu.make_async_copy(m_hbm.at[isrc.at[j]], bufs[k],
                                  gsems[k]).wait()

        def scat(j, k):
            pltpu.async_copy(bufs[k], acc.at[idst.at[j]], ssems[k], add=True)

        def scat_wait(j, k):
            pltpu.make_async_copy(bufs[k], acc.at[idst.at[j]],
                                  ssems[k]).wait()

        nb = 8
        for k in range(nb):
            gather(k, k)

        @pl.loop(0, ROWS // nb)
        def _(jj):
            j = jj * nb
            for k in range(nb):
                gather_wait(j + k, k)
                scat(j + k, k)
                kp = (k + nb // 2) % nb
                if k < nb // 2:
                    @pl.when(jj > 0)
                    def _():
                        scat_wait(j + k - nb // 2, kp)
                        gather(j + k + nb // 2, kp)
                else:
                    scat_wait(j + k - nb // 2, kp)

                    @pl.when(jj < ROWS // nb - 1)
                    def _():
                        gather(j + k + nb // 2, kp)

        for k in range(nb // 2, nb):
            scat_wait(ROWS - nb + k, k)
        plsc.subcore_barrier()

        @pl.when(cid == 0)
        def _():
            pltpu.sync_copy(acc.at[pl.ds(sid * SLICE, SLICE)],
                            out0_hbm.at[pl.ds(sid * SLICE, SLICE)])

        @pl.when(cid == 1)
        def _():
            pltpu.sync_copy(acc.at[pl.ds(sid * SLICE, SLICE)],
                            out1_hbm.at[pl.ds(sid * SLICE, SLICE)])

    @functools.partial(
        pl.kernel,
        mesh=mesh,
        out_type=jax.ShapeDtypeStruct((P,), jnp.float32),
        compiler_params=sc_params_tc_tiled,
        scratch_types=[
            pltpu.VMEM((PPT,), jnp.int32),
            pltpu.VMEM((PPT,), jnp.int32),
            pltpu.VMEM((2 * NR,), jnp.float32),
            pltpu.VMEM((PPT,), jnp.float32),
        ],
    )
    def pairs_kernel(pv_hbm, w_hbm, out_hbm, pa_v, pb_v, w_v, out_v):
        cid = lax.axis_index("c")
        sid = lax.axis_index("s")
        g = cid * NS + sid
        pltpu.sync_copy(w_hbm, w_v)
        pltpu.sync_copy(pv_hbm.at[pl.ds(g * PPT, PPT)], pa_v)
        pltpu.sync_copy(pv_hbm.at[pl.ds(PPAD + g * PPT, PPT)], pb_v)

        @pl.loop(0, PPT // 16)
        def _(i):
            ia = pa_v[pl.ds(i * 16, 16)]
            ib = pb_v[pl.ds(i * 16, 16)] + NR
            va = plsc.load_gather(w_v, [ia])
            vb = plsc.load_gather(w_v, [ib])
            out_v[pl.ds(i * 16, 16)] = va + vb

        @pl.when(g < TILES - 1)
        def _():
            pltpu.sync_copy(out_v, out_hbm.at[pl.ds(g * PPT, PPT)])

        @pl.when(g == TILES - 1)
        def _():
            pltpu.sync_copy(out_v.at[pl.ds(0, P - (TILES - 1) * PPT)],
                            out_hbm.at[pl.ds(g * PPT, P - (TILES - 1) * PPT)])

    return deg_kernel, scatter_kernel, pairs_kernel


BLK = 5120


def _dinv(d0_ref, d1_ref):
    return lax.rsqrt(d0_ref[...] + d1_ref[...] + 1.0)[:, None]


def _k1_body(d0_ref, d1_ref, x_ref, w_ref, m_ref):
    h = jnp.dot(x_ref[...], w_ref[...], preferred_element_type=jnp.float32)
    m_ref[...] = _dinv(d0_ref, d1_ref) * h


def _k2_body(d0_ref, d1_ref, p0_ref, p1_ref, m_ref, b_ref, w_ref, out_ref):
    dinv = _dinv(d0_ref, d1_ref)
    s = p0_ref[...] + p1_ref[...] + m_ref[...]
    h = jnp.maximum(dinv * s + b_ref[...], 0.0)
    out_ref[...] = dinv * jnp.dot(h, w_ref[...],
                                  preferred_element_type=jnp.float32)


def _k3_body(d0_ref, d1_ref, p0_ref, p1_ref, m_ref, b_ref, w_ref, bias_ref,
             z_ref):
    dinv = _dinv(d0_ref, d1_ref)
    s = p0_ref[...] + p1_ref[...] + m_ref[...]
    h = jnp.maximum(dinv * s + b_ref[...], 0.0)
    z = jnp.dot(h, w_ref[...], preferred_element_type=jnp.float32)
    z_ref[...] = z.T + bias_ref[...]


def _row_spec(w):
    return pl.BlockSpec((BLK, w), lambda i: (i, 0))


def _vec_spec():
    return pl.BlockSpec((BLK,), lambda i: (i,))


def _full_spec(r, c):
    return pl.BlockSpec((r, c), lambda i: (0, 0))


_GRID = NR // BLK

_k1 = pl.pallas_call(
    _k1_body,
    grid=(_GRID,),
    in_specs=[_vec_spec(), _vec_spec(), _row_spec(D_IN),
              _full_spec(D_IN, HID)],
    out_specs=_row_spec(HID),
    out_shape=jax.ShapeDtypeStruct((NR, HID), jnp.float32),
)

_k2 = pl.pallas_call(
    _k2_body,
    grid=(_GRID,),
    in_specs=[_vec_spec(), _vec_spec(), _row_spec(HID), _row_spec(HID),
              _row_spec(HID), _full_spec(1, HID), _full_spec(HID, HID)],
    out_specs=_row_spec(HID),
    out_shape=jax.ShapeDtypeStruct((NR, HID), jnp.float32),
)

_k3 = pl.pallas_call(
    _k3_body,
    grid=(_GRID,),
    in_specs=[_vec_spec(), _vec_spec(), _row_spec(HID), _row_spec(HID),
              _row_spec(HID), _full_spec(1, HID), _full_spec(HID, 2),
              _full_spec(2, 1)],
    out_specs=pl.BlockSpec((2, BLK), lambda i: (0, i)),
    out_shape=jax.ShapeDtypeStruct((2, NR), jnp.float32),
)


def kernel(x, edge_index, batch, pairs, W1, b1, W2, b2, Wh, bh):
    del batch
    i32 = jnp.int32
    f32 = jnp.float32
    deg_kernel, scatter_kernel, pairs_kernel = _sc_kernels()

    pad_idx = (jnp.arange(EPAD - E, dtype=i32) % (NR - N)) + N
    pad2 = jnp.broadcast_to(pad_idx, (2, EPAD - E))
    edges = jnp.concatenate([edge_index.astype(i32), pad2],
                            axis=1).reshape(2, TILES * ROWS, CH)
    edges = lax.optimization_barrier(edges)

    xp = jnp.pad(x.astype(f32), ((0, NR - N), (0, 0)))
    ones_deg = jnp.ones((CH,), f32)
    zeros_deg = jnp.zeros((NR,), f32)
    zeros_h = jnp.zeros((NR, HID), f32)

    pvt = jnp.concatenate([
        jnp.pad(pairs[:, 0].astype(i32), (0, PPAD - P)),
        jnp.pad(pairs[:, 1].astype(i32), (0, PPAD - P)),
    ])

    wh2 = jnp.stack([Wh[:HID, 0], Wh[HID:, 0]], axis=1).astype(f32)
    bias2 = jnp.concatenate([jnp.zeros((1,), f32), bh.astype(f32)])[:, None]

    d0, d1 = deg_kernel(edges, ones_deg, zeros_deg)
    m1 = _k1(d0, d1, xp, W1.astype(f32))
    p0, p1 = scatter_kernel(edges, m1, zeros_h)
    m2 = _k2(d0, d1, p0, p1, m1, b1.astype(f32)[None, :], W2.astype(f32))
    q0, q1 = scatter_kernel(edges, m2, zeros_h)
    z = _k3(d0, d1, q0, q1, m2, b2.astype(f32)[None, :], wh2, bias2)
    return pairs_kernel(pvt, z.reshape(2 * NR))
